# speedup vs baseline: 4.7869x; 4.7869x over previous
import jax
import jax.numpy as jnp
from jax.experimental import pallas as pl
from jax.experimental.pallas import tpu as pltpu

NCODES = 512


def _body(z_ref, w_ref, soft_ref, hard_ref, idx_ref):
    p = z_ref[0]
    w = w_ref[0]
    d = jnp.abs(p - w)
    m = jnp.min(d, axis=1, keepdims=True)
    e = jnp.exp(m - d)
    denom = jnp.sum(e, axis=1, keepdims=True)
    num = jnp.sum(e * w, axis=1, keepdims=True)
    soft_ref[0] = num / denom

    kidx = jax.lax.broadcasted_iota(jnp.int32, d.shape, 1)
    ismin = d == m
    idx = jnp.min(jnp.where(ismin, kidx, NCODES), axis=1, keepdims=True)
    idx_ref[0] = idx
    onehot = kidx == idx
    hard_ref[0] = jnp.sum(jnp.where(onehot, w, 0.0), axis=1, keepdims=True)


def kernel(z, codes):
    B, C, H, W = z.shape
    K = codes.shape[1]
    N = B * H * W
    z3 = z.reshape(C, N, 1)
    codes3 = codes.reshape(C, 1, K)

    soft, hard, idx = pl.pallas_call(
        _body,
        grid=(C,),
        in_specs=[
            pl.BlockSpec((1, N, 1), lambda c: (c, 0, 0)),
            pl.BlockSpec((1, 1, K), lambda c: (c, 0, 0)),
        ],
        out_specs=[
            pl.BlockSpec((1, N, 1), lambda c: (c, 0, 0)),
            pl.BlockSpec((1, N, 1), lambda c: (c, 0, 0)),
            pl.BlockSpec((1, N, 1), lambda c: (c, 0, 0)),
        ],
        out_shape=[
            jax.ShapeDtypeStruct((C, N, 1), jnp.float32),
            jax.ShapeDtypeStruct((C, N, 1), jnp.float32),
            jax.ShapeDtypeStruct((C, N, 1), jnp.int32),
        ],
        compiler_params=pltpu.CompilerParams(
            dimension_semantics=("parallel",),
        ),
    )(z3, codes3)

    def to_out(a):
        return a.reshape(C, N).T.reshape(B, H, W, C)

    return (to_out(soft), to_out(hard), to_out(idx))
